# Initial kernel scaffold; baseline (speedup 1.0000x reference)
import jax
import jax.numpy as jnp
from jax.experimental import pallas as pl


def kernel(neigh_vecs, self_vecs, edge_index, edge_weight, neigh_weights, self_weights):
    raise NotImplementedError("write your pallas kernel here")


# speedup vs baseline: 4.5116x; 4.5116x over previous
import functools

import jax
import jax.numpy as jnp
from jax import lax
from jax.experimental import pallas as pl
from jax.experimental.pallas import tpu as pltpu
from jax.experimental.pallas import tpu_sc as plsc

N = 10000
NP = 10240
E = 320000
D = 128

NC = 2
NS = 16
NW = NC * NS
EPW = E // NW
CH = 80
NCHUNK = EPW // CH
RPT = NP // NS
ZROWS = 128
ZCOPIES = RPT // ZROWS


def _sc_segment_sum(neigh_vecs, src, dst, ew):
    @functools.partial(
        pl.kernel,
        out_type=(
            jax.ShapeDtypeStruct((NP, D), jnp.float32),
            jax.ShapeDtypeStruct((NP, D), jnp.float32),
        ),
        mesh=plsc.VectorSubcoreMesh(core_axis_name="c", subcore_axis_name="s"),
        scratch_types=[
            pltpu.VMEM_SHARED((NP, D), jnp.float32),
            pltpu.VMEM((CH,), jnp.int32),
            pltpu.VMEM((CH,), jnp.int32),
            pltpu.VMEM((CH,), jnp.float32),
            pltpu.VMEM((CH, D), jnp.float32),
            pltpu.VMEM((ZROWS, D), jnp.float32),
            pltpu.SemaphoreType.DMA,
        ],
    )
    def body(neigh, src_h, dst_h, ew_h, p0, p1, acc, src_v, dst_v, w_v,
             rows_v, zbuf, sem):
        cid = lax.axis_index("c")
        sid = lax.axis_index("s")

        @pl.loop(0, ZROWS)
        def _zero(r):
            for j in range(D // 16):
                zbuf[r, pl.ds(j * 16, 16)] = jnp.zeros((16,), jnp.float32)

        for kk in range(ZCOPIES):
            pltpu.sync_copy(zbuf, acc.at[pl.ds(sid * RPT + kk * ZROWS, ZROWS)])
        plsc.subcore_barrier()

        wid = sid * NC + cid
        base = wid * EPW

        @pl.loop(0, NCHUNK)
        def _chunk(k):
            off = base + k * CH
            pltpu.sync_copy(src_h.at[pl.ds(off, CH)], src_v)
            pltpu.sync_copy(dst_h.at[pl.ds(off, CH)], dst_v)
            pltpu.sync_copy(ew_h.at[pl.ds(off, CH)], w_v)
            pltpu.async_copy(neigh.at[src_v], rows_v, sem).wait()

            @pl.loop(0, CH // 16)
            def _scale(g):
                wv = w_v[pl.ds(g * 16, 16)]
                for l in range(16):
                    w = wv[l]
                    e = g * 16 + l
                    for j in range(D // 16):
                        sl = pl.ds(j * 16, 16)
                        rows_v[e, sl] = rows_v[e, sl] * w

            pltpu.sync_copy(rows_v, acc.at[dst_v], add=True)

        plsc.subcore_barrier()

        r0 = sid * RPT

        @pl.when(cid == 0)
        def _():
            pltpu.sync_copy(acc.at[pl.ds(r0, RPT)], p0.at[pl.ds(r0, RPT)])

        @pl.when(cid == 1)
        def _():
            pltpu.sync_copy(acc.at[pl.ds(r0, RPT)], p1.at[pl.ds(r0, RPT)])

    return body(neigh_vecs, src, dst, ew)


BN = 1000


def _tc_body(self_ref, p0_ref, p1_ref, sw_ref, nw_ref, out_ref):
    fs = jnp.dot(self_ref[...], sw_ref[...], preferred_element_type=jnp.float32)
    nm = p0_ref[...] + p1_ref[...]
    fn = jnp.dot(nm, nw_ref[...], preferred_element_type=jnp.float32)
    out_ref[:, :D] = jnp.maximum(fs, 0.0)
    out_ref[:, D:] = jnp.maximum(fn, 0.0)


def _tc_dense(self_vecs, p0, p1, self_weights, neigh_weights):
    return pl.pallas_call(
        _tc_body,
        grid=(N // BN,),
        in_specs=[
            pl.BlockSpec((BN, D), lambda i: (i, 0)),
            pl.BlockSpec((BN, D), lambda i: (i, 0)),
            pl.BlockSpec((BN, D), lambda i: (i, 0)),
            pl.BlockSpec((D, D), lambda i: (0, 0)),
            pl.BlockSpec((D, D), lambda i: (0, 0)),
        ],
        out_specs=pl.BlockSpec((BN, 2 * D), lambda i: (i, 0)),
        out_shape=jax.ShapeDtypeStruct((N, 2 * D), jnp.float32),
    )(self_vecs, p0, p1, self_weights, neigh_weights)


def kernel(neigh_vecs, self_vecs, edge_index, edge_weight, neigh_weights,
           self_weights):
    src = edge_index[0].astype(jnp.int32)
    dst = edge_index[1].astype(jnp.int32)
    p0, p1 = _sc_segment_sum(neigh_vecs, src, dst, edge_weight)
    return _tc_dense(self_vecs, p0, p1, self_weights, neigh_weights)
